# baseline (speedup 1.0000x reference)
import functools

import jax
import jax.numpy as jnp
from jax import lax
from jax.experimental import pallas as pl
from jax.experimental.pallas import tpu as pltpu
from jax.experimental.pallas import tpu_sc as plsc

HIDDEN = 64
OUT_DIM = 64
EPS = 1e-12

_NC = 2
_NS = 16
_NW = _NC * _NS

_GCHUNK = 128
_KFIRE = 5


def _make_sc_gather(n_pairs, n_tok):
    per_w = n_tok // _NW
    step = _KFIRE * _GCHUNK
    n_outer = per_w // step
    assert per_w % step == 0
    mesh = plsc.VectorSubcoreMesh(core_axis_name="c", subcore_axis_name="s")

    @functools.partial(
        pl.kernel,
        mesh=mesh,
        out_type=jax.ShapeDtypeStruct((n_tok, 128), jnp.float32),
        scratch_types=[
            pltpu.VMEM((_KFIRE * _GCHUNK,), jnp.int32),
            pltpu.VMEM((step, 128), jnp.float32),
            pltpu.SemaphoreType.DMA,
        ],
        compiler_params=pltpu.CompilerParams(use_tc_tiling_on_sc=True),
    )
    def sc_gather(t2_hbm, idx_hbm, out_hbm, idx_v, rows_v, gsem):
        wid = lax.axis_index("s") * _NC + lax.axis_index("c")
        base = wid * per_w

        def body(it, carry):
            off = base + it * step
            pltpu.sync_copy(idx_hbm.at[pl.ds(off, step)], idx_v)
            starts = [
                pltpu.async_copy(
                    t2_hbm.at[idx_v.at[pl.ds(j * _GCHUNK, _GCHUNK)]],
                    rows_v.at[pl.ds(j * _GCHUNK, _GCHUNK)],
                    gsem,
                )
                for j in range(_KFIRE)
            ]
            for h in starts:
                h.wait()
            pltpu.sync_copy(rows_v, out_hbm.at[pl.ds(off, step)])
            return carry

        lax.fori_loop(0, n_outer, body, 0)

    return sc_gather


def _ln_proj_body(e2_ref, sel_ref, lnw_ref, lnb_ref, wt_ref, b_ref, o_ref):
    e2 = e2_ref[...]
    sel = sel_ref[...]
    e = jnp.where(sel > 0.5, e2[:, HIDDEN:], e2[:, :HIDDEN])
    u = jnp.mean(e, axis=1, keepdims=True)
    d = e - u
    s = jnp.mean(d * d, axis=1, keepdims=True)
    x = d * lax.rsqrt(s + EPS)
    x = x * lnw_ref[...] + lnb_ref[...]
    o_ref[...] = (
        jnp.dot(x, wt_ref[...], preferred_element_type=jnp.float32) + b_ref[...]
    )


def _ln_proj(e2, sel, ln_weight, ln_bias, fc_w, fc_b, block_rows=2048):
    n = e2.shape[0]
    h = HIDDEN
    assert n % block_rows == 0
    grid = (n // block_rows,)
    return pl.pallas_call(
        _ln_proj_body,
        grid=grid,
        in_specs=[
            pl.BlockSpec((block_rows, 2 * h), lambda i: (i, 0)),
            pl.BlockSpec((block_rows, 1), lambda i: (i, 0)),
            pl.BlockSpec((1, h), lambda i: (0, 0)),
            pl.BlockSpec((1, h), lambda i: (0, 0)),
            pl.BlockSpec((h, OUT_DIM), lambda i: (0, 0)),
            pl.BlockSpec((1, OUT_DIM), lambda i: (0, 0)),
        ],
        out_specs=pl.BlockSpec((block_rows, OUT_DIM), lambda i: (i, 0)),
        out_shape=jax.ShapeDtypeStruct((n, OUT_DIM), jnp.float32),
    )(e2, sel, ln_weight.reshape(1, h), ln_bias.reshape(1, h),
      fc_w.T, fc_b.reshape(1, OUT_DIM))


def kernel(flat_input_ids, tag_table, ln_weight, ln_bias, fc_w, fc_b):
    b, l = flat_input_ids.shape
    vocab, h = tag_table.shape
    n_tok = b * l
    idx = flat_input_ids.reshape(-1).astype(jnp.int32)
    idx2 = idx >> 1
    sel = (idx & 1).astype(jnp.float32).reshape(n_tok, 1)
    t2 = tag_table.reshape(vocab // 2, 2 * h)
    gather = _make_sc_gather(vocab // 2, n_tok)
    e2 = gather(t2, idx2)
    out = _ln_proj(e2, sel, ln_weight, ln_bias, fc_w, fc_b)
    return out.reshape(b, l, OUT_DIM)


# speedup vs baseline: 1.9051x; 1.9051x over previous
import functools

import jax
import jax.numpy as jnp
from jax import lax
from jax.experimental import pallas as pl
from jax.experimental.pallas import tpu as pltpu
from jax.experimental.pallas import tpu_sc as plsc

HIDDEN = 64
OUT_DIM = 64
EPS = 1e-12

_K = 512000
_AC = 2048
_NLO = _K // _AC
_NHI = _NLO

_NC = 2
_NS = 16
_NW = _NC * _NS

_GCHUNK = 128
_KFIRE = 5


def _table_g_body(xlo_ref, xhi_ref, wl_ref, bl_ref, o_ref):
    wl = wl_ref[...]
    bl = bl_ref[...]

    def g(x):
        u = jnp.mean(x, axis=0, keepdims=True)
        d = x - u
        s = jnp.mean(d * d, axis=0, keepdims=True)
        xn = d * lax.rsqrt(s + EPS)
        return jnp.dot(wl, xn, preferred_element_type=jnp.float32) + bl

    o_ref[:, 0:HIDDEN] = jnp.transpose(g(xlo_ref[...]), (1, 0))
    o_ref[:, HIDDEN:2 * HIDDEN] = jnp.transpose(g(xhi_ref[...]), (1, 0))


def _table_g(tableT, wl, bl):
    h, vocab = tableT.shape
    nhi_last = (vocab - _K - 1) // _AC
    return pl.pallas_call(
        _table_g_body,
        grid=(_NLO,),
        in_specs=[
            pl.BlockSpec((h, _AC), lambda i: (0, i)),
            pl.BlockSpec((h, _AC),
                         lambda i: (0, _NLO + jnp.minimum(i, nhi_last))),
            pl.BlockSpec((h, h), lambda i: (0, 0)),
            pl.BlockSpec((h, 1), lambda i: (0, 0)),
        ],
        out_specs=pl.BlockSpec((_AC, 2 * h), lambda i: (i, 0)),
        out_shape=jax.ShapeDtypeStruct((_K, 2 * h), jnp.float32),
    )(tableT, tableT, wl, bl)


def _make_sc_gather(n_tok):
    per_w = n_tok // _NW
    step = _KFIRE * _GCHUNK
    n_outer = per_w // step
    assert per_w % step == 0
    mesh = plsc.VectorSubcoreMesh(core_axis_name="c", subcore_axis_name="s")

    @functools.partial(
        pl.kernel,
        mesh=mesh,
        out_type=jax.ShapeDtypeStruct((n_tok, 128), jnp.float32),
        scratch_types=[
            pltpu.VMEM((_KFIRE * _GCHUNK,), jnp.int32),
            pltpu.VMEM((step, 128), jnp.float32),
            pltpu.SemaphoreType.DMA,
        ],
        compiler_params=pltpu.CompilerParams(use_tc_tiling_on_sc=True),
    )
    def sc_gather(g3_hbm, idx_hbm, out_hbm, idx_v, rows_v, gsem):
        wid = lax.axis_index("s") * _NC + lax.axis_index("c")
        base = wid * per_w

        def body(it, carry):
            off = base + it * step
            pltpu.sync_copy(idx_hbm.at[pl.ds(off, step)], idx_v)
            starts = [
                pltpu.async_copy(
                    g3_hbm.at[idx_v.at[pl.ds(j * _GCHUNK, _GCHUNK)]],
                    rows_v.at[pl.ds(j * _GCHUNK, _GCHUNK)],
                    gsem,
                )
                for j in range(_KFIRE)
            ]
            for hdl in starts:
                hdl.wait()
            pltpu.sync_copy(rows_v, out_hbm.at[pl.ds(off, step)])
            return carry

        lax.fori_loop(0, n_outer, body, 0)

    return sc_gather


def _sel_t_body(e_ref, sel_ref, o_ref):
    e = e_ref[...]
    sel = sel_ref[...]
    x = jnp.where(sel[:, :, None] > 0.5,
                  e[:, :, HIDDEN:2 * HIDDEN], e[:, :, 0:HIDDEN])
    o_ref[...] = jnp.transpose(x, (0, 2, 1))


def _sel_transpose(e3, sel2, bb=256):
    l, b, w = e3.shape
    assert b % bb == 0
    return pl.pallas_call(
        _sel_t_body,
        grid=(b // bb,),
        in_specs=[
            pl.BlockSpec((l, bb, w), lambda i: (0, i, 0)),
            pl.BlockSpec((l, bb), lambda i: (0, i)),
        ],
        out_specs=pl.BlockSpec((l, OUT_DIM, bb), lambda i: (0, 0, i)),
        out_shape=jax.ShapeDtypeStruct((l, OUT_DIM, b), jnp.float32),
    )(e3, sel2)


def kernel(flat_input_ids, tag_table, ln_weight, ln_bias, fc_w, fc_b):
    b, l = flat_input_ids.shape
    vocab, h = tag_table.shape
    n_tok = b * l

    ids = flat_input_ids.astype(jnp.int32)
    sel2 = (ids >= _K).astype(jnp.float32).T
    idx_eff = jnp.where(ids < _K, ids, ids - _K)
    idx_perm = idx_eff.T.reshape(-1)

    wl = fc_w * ln_weight[None, :]
    bl = (fc_w @ ln_bias + fc_b).reshape(h, 1)

    g3 = _table_g(tag_table.T, wl, bl)
    e_g = _make_sc_gather(n_tok)(g3, idx_perm)
    e3 = e_g.reshape(l, b, 128)
    out_t = _sel_transpose(e3, sel2)
    return out_t.transpose(2, 0, 1)


# speedup vs baseline: 2.0288x; 1.0649x over previous
import functools

import jax
import jax.numpy as jnp
from jax import lax
from jax.experimental import pallas as pl
from jax.experimental.pallas import tpu as pltpu
from jax.experimental.pallas import tpu_sc as plsc

HIDDEN = 64
OUT_DIM = 64
EPS = 1e-12

_K = 512000
_AC = 2048
_NLO = _K // _AC
_NHI = _NLO

_NC = 2
_NS = 16
_NW = _NC * _NS

_GCHUNK = 128
_KFIRE = 5


def _table_g_body(xlo_ref, xhi_ref, wl_ref, blr_ref, o_ref):
    wl = wl_ref[...]
    blr = blr_ref[...]

    def g(x):
        u = jnp.mean(x, axis=0, keepdims=True)
        d = x - u
        s = jnp.mean(d * d, axis=0, keepdims=True)
        xn = d * lax.rsqrt(s + EPS)
        return lax.dot_general(
            xn, wl, (((0,), (1,)), ((), ())),
            preferred_element_type=jnp.float32,
        ) + blr

    o_ref[:, 0:HIDDEN] = g(xlo_ref[...])
    o_ref[:, HIDDEN:2 * HIDDEN] = g(xhi_ref[...])


def _table_g(tableT, wl, bl):
    h, vocab = tableT.shape
    nhi_last = (vocab - _K - 1) // _AC
    return pl.pallas_call(
        _table_g_body,
        grid=(_NLO,),
        in_specs=[
            pl.BlockSpec((h, _AC), lambda i: (0, i)),
            pl.BlockSpec((h, _AC),
                         lambda i: (0, _NLO + jnp.minimum(i, nhi_last))),
            pl.BlockSpec((h, h), lambda i: (0, 0)),
            pl.BlockSpec((1, h), lambda i: (0, 0)),
        ],
        out_specs=pl.BlockSpec((_AC, 2 * h), lambda i: (i, 0)),
        out_shape=jax.ShapeDtypeStruct((_K, 2 * h), jnp.float32),
    )(tableT, tableT, wl, bl)


def _make_sc_gather(n_tok):
    per_w = n_tok // _NW
    step = _KFIRE * _GCHUNK
    n_outer = per_w // step
    assert per_w % step == 0
    mesh = plsc.VectorSubcoreMesh(core_axis_name="c", subcore_axis_name="s")

    @functools.partial(
        pl.kernel,
        mesh=mesh,
        out_type=jax.ShapeDtypeStruct((n_tok, 128), jnp.float32),
        scratch_types=[
            pltpu.VMEM((_KFIRE * _GCHUNK,), jnp.int32),
            pltpu.VMEM((step, 128), jnp.float32),
            pltpu.SemaphoreType.DMA,
        ],
        compiler_params=pltpu.CompilerParams(use_tc_tiling_on_sc=True),
    )
    def sc_gather(g3_hbm, idx_hbm, out_hbm, idx_v, rows_v, gsem):
        wid = lax.axis_index("s") * _NC + lax.axis_index("c")
        base = wid * per_w

        def body(it, carry):
            off = base + it * step
            pltpu.sync_copy(idx_hbm.at[pl.ds(off, step)], idx_v)
            starts = [
                pltpu.async_copy(
                    g3_hbm.at[idx_v.at[pl.ds(j * _GCHUNK, _GCHUNK)]],
                    rows_v.at[pl.ds(j * _GCHUNK, _GCHUNK)],
                    gsem,
                )
                for j in range(_KFIRE)
            ]
            for hdl in starts:
                hdl.wait()
            pltpu.sync_copy(rows_v, out_hbm.at[pl.ds(off, step)])
            return carry

        lax.fori_loop(0, n_outer, body, 0)

    return sc_gather


def _sel_t_body(e_ref, sel_ref, o_ref):
    e = e_ref[...]
    sel = sel_ref[...]
    x = jnp.where(sel[:, :, None] > 0.5,
                  e[:, :, HIDDEN:2 * HIDDEN], e[:, :, 0:HIDDEN])
    o_ref[...] = jnp.transpose(x, (0, 2, 1))


def _sel_transpose(e3, sel2, bb=256):
    l, b, w = e3.shape
    assert b % bb == 0
    return pl.pallas_call(
        _sel_t_body,
        grid=(b // bb,),
        in_specs=[
            pl.BlockSpec((l, bb, w), lambda i: (0, i, 0)),
            pl.BlockSpec((l, bb), lambda i: (0, i)),
        ],
        out_specs=pl.BlockSpec((l, OUT_DIM, bb), lambda i: (0, 0, i)),
        out_shape=jax.ShapeDtypeStruct((l, OUT_DIM, b), jnp.float32),
    )(e3, sel2)


def kernel(flat_input_ids, tag_table, ln_weight, ln_bias, fc_w, fc_b):
    b, l = flat_input_ids.shape
    vocab, h = tag_table.shape
    n_tok = b * l

    ids = flat_input_ids.astype(jnp.int32)
    sel2 = (ids >= _K).astype(jnp.float32).T
    idx_eff = jnp.where(ids < _K, ids, ids - _K)
    idx_perm = idx_eff.T.reshape(-1)

    wl = fc_w * ln_weight[None, :]
    bl = (fc_w @ ln_bias + fc_b).reshape(1, h)

    g3 = _table_g(tag_table.T, wl, bl)
    e_g = _make_sc_gather(n_tok)(g3, idx_perm)
    e3 = e_g.reshape(l, b, 128)
    out_t = _sel_transpose(e3, sel2)
    return out_t.transpose(2, 0, 1)


# speedup vs baseline: 2.5067x; 1.2356x over previous
import functools

import jax
import jax.numpy as jnp
from jax import lax
from jax.experimental import pallas as pl
from jax.experimental.pallas import tpu as pltpu
from jax.experimental.pallas import tpu_sc as plsc

HIDDEN = 64
OUT_DIM = 64
EPS = 1e-12

_Q = 256000
_AC = 2048
_NQB = _Q // _AC

_NC = 2
_NS = 16
_NW = _NC * _NS

_GCHUNK = 128
_KFIRE = 5


def _table_g_body(x0_ref, x1_ref, x2_ref, x3_ref, wl_ref, blr_ref, o_ref):
    wl = wl_ref[...]
    blr = blr_ref[...]

    def gbits(x):
        u = jnp.mean(x, axis=0, keepdims=True)
        d = x - u
        s = jnp.mean(d * d, axis=0, keepdims=True)
        xn = d * lax.rsqrt(s + EPS)
        y = lax.dot_general(
            xn, wl, (((0,), (1,)), ((), ())),
            preferred_element_type=jnp.float32,
        ) + blr
        t = lax.bitcast_convert_type(y, jnp.int32)
        t = t + 0x7FFF + ((t >> 16) & 1)
        return (t >> 16) & 0xFFFF

    o_ref[:, 0:HIDDEN] = gbits(x0_ref[...]) | (gbits(x2_ref[...]) << 16)
    o_ref[:, HIDDEN:2 * HIDDEN] = gbits(x1_ref[...]) | (gbits(x3_ref[...]) << 16)


def _table_g(tableT, wl, bl):
    h, vocab = tableT.shape
    n3_last = (vocab - 3 * _Q - 1) // _AC
    return pl.pallas_call(
        _table_g_body,
        grid=(_NQB,),
        in_specs=[
            pl.BlockSpec((h, _AC), lambda i: (0, i)),
            pl.BlockSpec((h, _AC), lambda i: (0, _NQB + i)),
            pl.BlockSpec((h, _AC), lambda i: (0, 2 * _NQB + i)),
            pl.BlockSpec((h, _AC),
                         lambda i: (0, 3 * _NQB + jnp.minimum(i, n3_last))),
            pl.BlockSpec((h, h), lambda i: (0, 0)),
            pl.BlockSpec((1, h), lambda i: (0, 0)),
        ],
        out_specs=pl.BlockSpec((_AC, 2 * h), lambda i: (i, 0)),
        out_shape=jax.ShapeDtypeStruct((_Q, 2 * h), jnp.int32),
    )(tableT, tableT, tableT, tableT, wl, bl)


def _make_sc_gather(n_tok):
    per_w = n_tok // _NW
    step = _KFIRE * _GCHUNK
    n_outer = per_w // step
    assert per_w % step == 0
    mesh = plsc.VectorSubcoreMesh(core_axis_name="c", subcore_axis_name="s")

    @functools.partial(
        pl.kernel,
        mesh=mesh,
        out_type=jax.ShapeDtypeStruct((n_tok, 128), jnp.int32),
        scratch_types=[
            pltpu.VMEM((_KFIRE * _GCHUNK,), jnp.int32),
            pltpu.VMEM((step, 128), jnp.int32),
            pltpu.SemaphoreType.DMA,
        ],
        compiler_params=pltpu.CompilerParams(use_tc_tiling_on_sc=True),
    )
    def sc_gather(g_hbm, idx_hbm, out_hbm, idx_v, rows_v, gsem):
        wid = lax.axis_index("s") * _NC + lax.axis_index("c")
        base = wid * per_w

        def body(it, carry):
            off = base + it * step
            pltpu.sync_copy(idx_hbm.at[pl.ds(off, step)], idx_v)
            starts = [
                pltpu.async_copy(
                    g_hbm.at[idx_v.at[pl.ds(j * _GCHUNK, _GCHUNK)]],
                    rows_v.at[pl.ds(j * _GCHUNK, _GCHUNK)],
                    gsem,
                )
                for j in range(_KFIRE)
            ]
            for hdl in starts:
                hdl.wait()
            pltpu.sync_copy(rows_v, out_hbm.at[pl.ds(off, step)])
            return carry

        lax.fori_loop(0, n_outer, body, 0)

    return sc_gather


def _sel_t_body(e_ref, q_ref, o_ref):
    e = e_ref[...]
    q = q_ref[...]
    q3 = q[:, :, None]
    lane_hi = (q3 & 1) == 1
    w = jnp.where(lane_hi, e[:, :, HIDDEN:2 * HIDDEN], e[:, :, 0:HIDDEN])
    word_hi = q3 >= 2
    bits = jnp.where(word_hi, w & jnp.int32(-65536), w << 16)
    x = lax.bitcast_convert_type(bits, jnp.float32)
    o_ref[...] = jnp.transpose(x, (0, 2, 1))


def _sel_transpose(e3, q2, bb=256):
    l, b, w = e3.shape
    assert b % bb == 0
    return pl.pallas_call(
        _sel_t_body,
        grid=(b // bb,),
        in_specs=[
            pl.BlockSpec((l, bb, w), lambda i: (0, i, 0)),
            pl.BlockSpec((l, bb), lambda i: (0, i)),
        ],
        out_specs=pl.BlockSpec((l, OUT_DIM, bb), lambda i: (0, 0, i)),
        out_shape=jax.ShapeDtypeStruct((l, OUT_DIM, b), jnp.float32),
    )(e3, q2)


def kernel(flat_input_ids, tag_table, ln_weight, ln_bias, fc_w, fc_b):
    b, l = flat_input_ids.shape
    vocab, h = tag_table.shape
    n_tok = b * l

    ids = flat_input_ids.astype(jnp.int32)
    q2 = (ids // _Q).T
    idx_perm = (ids % _Q).T.reshape(-1)

    wl = fc_w * ln_weight[None, :]
    bl = (fc_w @ ln_bias + fc_b).reshape(1, h)

    g = _table_g(tag_table.T, wl, bl)
    e_g = _make_sc_gather(n_tok)(g, idx_perm)
    e3 = e_g.reshape(l, b, 128)
    out_t = _sel_transpose(e3, q2)
    return out_t.transpose(2, 0, 1)


# speedup vs baseline: 2.5766x; 1.0279x over previous
import functools

import jax
import jax.numpy as jnp
from jax import lax
from jax.experimental import pallas as pl
from jax.experimental.pallas import tpu as pltpu
from jax.experimental.pallas import tpu_sc as plsc

HIDDEN = 64
OUT_DIM = 64
EPS = 1e-12

_Q = 256000
_AC = 5120
_NQB = _Q // _AC

_NC = 2
_NS = 16
_NW = _NC * _NS

_GSTEP = 320


def _table_g_body(x0_ref, x1_ref, x2_ref, x3_ref, wl_ref, blr_ref, o_ref):
    wl = wl_ref[...]
    blr = blr_ref[...]

    def gbits(x):
        u = jnp.mean(x, axis=0, keepdims=True)
        d = x - u
        s = jnp.mean(d * d, axis=0, keepdims=True)
        xn = d * lax.rsqrt(s + EPS)
        y = lax.dot_general(
            xn, wl, (((0,), (1,)), ((), ())),
            preferred_element_type=jnp.float32,
        ) + blr
        t = lax.bitcast_convert_type(y, jnp.int32)
        t = t + 0x7FFF + ((t >> 16) & 1)
        return (t >> 16) & 0xFFFF

    o_ref[:, 0:HIDDEN] = gbits(x0_ref[...]) | (gbits(x2_ref[...]) << 16)
    o_ref[:, HIDDEN:2 * HIDDEN] = gbits(x1_ref[...]) | (gbits(x3_ref[...]) << 16)


def _table_g(tableT, wl, bl):
    h, vocab = tableT.shape
    n3_last = (vocab - 3 * _Q - 1) // _AC
    return pl.pallas_call(
        _table_g_body,
        grid=(_NQB,),
        in_specs=[
            pl.BlockSpec((h, _AC), lambda i: (0, i)),
            pl.BlockSpec((h, _AC), lambda i: (0, _NQB + i)),
            pl.BlockSpec((h, _AC), lambda i: (0, 2 * _NQB + i)),
            pl.BlockSpec((h, _AC),
                         lambda i: (0, 3 * _NQB + jnp.minimum(i, n3_last))),
            pl.BlockSpec((h, h), lambda i: (0, 0)),
            pl.BlockSpec((1, h), lambda i: (0, 0)),
        ],
        out_specs=pl.BlockSpec((_AC, 2 * h), lambda i: (i, 0)),
        out_shape=jax.ShapeDtypeStruct((_Q, 2 * h), jnp.int32),
    )(tableT, tableT, tableT, tableT, wl, bl)


def _make_sc_gather(n_tok):
    per_w = n_tok // _NW
    step = _GSTEP
    n_outer = per_w // step
    assert per_w % step == 0
    mesh = plsc.VectorSubcoreMesh(core_axis_name="c", subcore_axis_name="s")

    @functools.partial(
        pl.kernel,
        mesh=mesh,
        out_type=jax.ShapeDtypeStruct((n_tok, 128), jnp.int32),
        scratch_types=[
            pltpu.VMEM((step,), jnp.int32),
            pltpu.VMEM((step,), jnp.int32),
            pltpu.VMEM((step, 128), jnp.int32),
            pltpu.VMEM((step, 128), jnp.int32),
            pltpu.SemaphoreType.DMA,
            pltpu.SemaphoreType.DMA,
        ],
        compiler_params=pltpu.CompilerParams(use_tc_tiling_on_sc=True),
    )
    def sc_gather(g_hbm, idx_hbm, out_hbm, idx_v0, idx_v1, rows_v0, rows_v1,
                  gsem, ssem):
        wid = lax.axis_index("s") * _NC + lax.axis_index("c")
        base = wid * per_w
        idx_bufs = (idx_v0, idx_v1)
        row_bufs = (rows_v0, rows_v1)
        pending = [None, None]
        for it in range(n_outer):
            p = it % 2
            off = base + it * step
            pltpu.sync_copy(idx_hbm.at[pl.ds(off, step)], idx_bufs[p])
            if pending[p] is not None:
                pending[p].wait()
            pltpu.async_copy(g_hbm.at[idx_bufs[p]], row_bufs[p], gsem).wait()
            pending[p] = pltpu.async_copy(
                row_bufs[p], out_hbm.at[pl.ds(off, step)], ssem)
        pending[0].wait()
        pending[1].wait()

    return sc_gather


def _sel_t_body(e_ref, q_ref, o_ref):
    e = e_ref[...]
    q = q_ref[...]
    q3 = q[:, :, None]
    lane_hi = (q3 & 1) == 1
    w = jnp.where(lane_hi, e[:, :, HIDDEN:2 * HIDDEN], e[:, :, 0:HIDDEN])
    word_hi = q3 >= 2
    bits = jnp.where(word_hi, w & jnp.int32(-65536), w << 16)
    x = lax.bitcast_convert_type(bits, jnp.float32)
    o_ref[...] = jnp.transpose(x, (0, 2, 1))


def _sel_transpose(e3, q2, bb=256):
    l, b, w = e3.shape
    assert b % bb == 0
    return pl.pallas_call(
        _sel_t_body,
        grid=(b // bb,),
        in_specs=[
            pl.BlockSpec((l, bb, w), lambda i: (0, i, 0)),
            pl.BlockSpec((l, bb), lambda i: (0, i)),
        ],
        out_specs=pl.BlockSpec((l, OUT_DIM, bb), lambda i: (0, 0, i)),
        out_shape=jax.ShapeDtypeStruct((l, OUT_DIM, b), jnp.float32),
    )(e3, q2)


def kernel(flat_input_ids, tag_table, ln_weight, ln_bias, fc_w, fc_b):
    b, l = flat_input_ids.shape
    vocab, h = tag_table.shape
    n_tok = b * l

    ids = flat_input_ids.astype(jnp.int32)
    q2 = (ids // _Q).T
    idx_perm = (ids % _Q).T.reshape(-1)

    wl = fc_w * ln_weight[None, :]
    bl = (fc_w @ ln_bias + fc_b).reshape(1, h)

    g = _table_g(tag_table.T, wl, bl)
    e_g = _make_sc_gather(n_tok)(g, idx_perm)
    e3 = e_g.reshape(l, b, 128)
    out_t = _sel_transpose(e3, q2)
    return out_t.transpose(2, 0, 1)


# speedup vs baseline: 2.5793x; 1.0011x over previous
import functools

import jax
import jax.numpy as jnp
from jax import lax
from jax.experimental import pallas as pl
from jax.experimental.pallas import tpu as pltpu
from jax.experimental.pallas import tpu_sc as plsc

HIDDEN = 64
OUT_DIM = 64
EPS = 1e-12

_Q = 256000
_AC = 5120
_NQB = _Q // _AC

_NC = 2
_NS = 16
_NW = _NC * _NS

_GSTEP = 320


def _table_g_body(x0_ref, x1_ref, x2_ref, x3_ref, wl_ref, blr_ref, o_ref):
    wl = wl_ref[...]
    blr = blr_ref[...]

    def gbits(x):
        u = jnp.mean(x, axis=0, keepdims=True)
        d = x - u
        s = jnp.mean(d * d, axis=0, keepdims=True)
        xn = d * lax.rsqrt(s + EPS)
        y = lax.dot_general(
            xn, wl, (((0,), (1,)), ((), ())),
            preferred_element_type=jnp.float32,
        ) + blr
        t = lax.bitcast_convert_type(y, jnp.int32)
        t = t + 0x7FFF + ((t >> 16) & 1)
        return (t >> 16) & 0xFFFF

    o_ref[:, 0:HIDDEN] = gbits(x0_ref[...]) | (gbits(x2_ref[...]) << 16)
    o_ref[:, HIDDEN:2 * HIDDEN] = gbits(x1_ref[...]) | (gbits(x3_ref[...]) << 16)


def _table_g(tableT, wl, bl):
    h, vocab = tableT.shape
    n3_last = (vocab - 3 * _Q - 1) // _AC
    return pl.pallas_call(
        _table_g_body,
        grid=(_NQB,),
        in_specs=[
            pl.BlockSpec((h, _AC), lambda i: (0, i)),
            pl.BlockSpec((h, _AC), lambda i: (0, _NQB + i)),
            pl.BlockSpec((h, _AC), lambda i: (0, 2 * _NQB + i)),
            pl.BlockSpec((h, _AC),
                         lambda i: (0, 3 * _NQB + jnp.minimum(i, n3_last))),
            pl.BlockSpec((h, h), lambda i: (0, 0)),
            pl.BlockSpec((1, h), lambda i: (0, 0)),
        ],
        out_specs=pl.BlockSpec((_AC, 2 * h), lambda i: (i, 0)),
        out_shape=jax.ShapeDtypeStruct((_Q, 2 * h), jnp.int32),
        compiler_params=pltpu.CompilerParams(fuse_transposed_lhs_in_matmul=True),
    )(tableT, tableT, tableT, tableT, wl, bl)


def _make_sc_gather(n_tok):
    per_w = n_tok // _NW
    step = _GSTEP
    n_outer = per_w // step
    assert per_w % step == 0
    mesh = plsc.VectorSubcoreMesh(core_axis_name="c", subcore_axis_name="s")

    @functools.partial(
        pl.kernel,
        mesh=mesh,
        out_type=jax.ShapeDtypeStruct((n_tok, 128), jnp.int32),
        scratch_types=[
            pltpu.VMEM((step,), jnp.int32),
            pltpu.VMEM((step,), jnp.int32),
            pltpu.VMEM((step, 128), jnp.int32),
            pltpu.VMEM((step, 128), jnp.int32),
            pltpu.SemaphoreType.DMA,
            pltpu.SemaphoreType.DMA,
        ],
        compiler_params=pltpu.CompilerParams(use_tc_tiling_on_sc=True),
    )
    def sc_gather(g_hbm, idx_hbm, out_hbm, idx_v0, idx_v1, rows_v0, rows_v1,
                  gsem, ssem):
        wid = lax.axis_index("s") * _NC + lax.axis_index("c")
        base = wid * per_w
        idx_bufs = (idx_v0, idx_v1)
        row_bufs = (rows_v0, rows_v1)
        pending = [None, None]
        for it in range(n_outer):
            p = it % 2
            off = base + it * step
            pltpu.sync_copy(idx_hbm.at[pl.ds(off, step)], idx_bufs[p])
            if pending[p] is not None:
                pending[p].wait()
            pltpu.async_copy(g_hbm.at[idx_bufs[p]], row_bufs[p], gsem).wait()
            pending[p] = pltpu.async_copy(
                row_bufs[p], out_hbm.at[pl.ds(off, step)], ssem)
        pending[0].wait()
        pending[1].wait()

    return sc_gather


def _sel_t_body(e_ref, q_ref, o_ref):
    e = e_ref[...]
    q = q_ref[...]
    q3 = q[:, :, None]
    lane_hi = (q3 & 1) == 1
    w = jnp.where(lane_hi, e[:, :, HIDDEN:2 * HIDDEN], e[:, :, 0:HIDDEN])
    word_hi = q3 >= 2
    bits = jnp.where(word_hi, w & jnp.int32(-65536), w << 16)
    x = lax.bitcast_convert_type(bits, jnp.float32)
    o_ref[...] = jnp.transpose(x, (0, 2, 1))


def _sel_transpose(e3, q2, bb=256):
    l, b, w = e3.shape
    assert b % bb == 0
    return pl.pallas_call(
        _sel_t_body,
        grid=(b // bb,),
        in_specs=[
            pl.BlockSpec((l, bb, w), lambda i: (0, i, 0)),
            pl.BlockSpec((l, bb), lambda i: (0, i)),
        ],
        out_specs=pl.BlockSpec((l, OUT_DIM, bb), lambda i: (0, 0, i)),
        out_shape=jax.ShapeDtypeStruct((l, OUT_DIM, b), jnp.float32),
    )(e3, q2)


def kernel(flat_input_ids, tag_table, ln_weight, ln_bias, fc_w, fc_b):
    b, l = flat_input_ids.shape
    vocab, h = tag_table.shape
    n_tok = b * l

    ids = flat_input_ids.astype(jnp.int32)
    q2 = (ids // _Q).T
    idx_perm = (ids % _Q).T.reshape(-1)

    wl = fc_w * ln_weight[None, :]
    bl = (fc_w @ ln_bias + fc_b).reshape(1, h)

    g = _table_g(tag_table.T, wl, bl)
    e_g = _make_sc_gather(n_tok)(g, idx_perm)
    e3 = e_g.reshape(l, b, 128)
    out_t = _sel_transpose(e3, q2)
    return out_t.transpose(2, 0, 1)


# speedup vs baseline: 2.6905x; 1.0431x over previous
import functools

import jax
import jax.numpy as jnp
from jax import lax
from jax.experimental import pallas as pl
from jax.experimental.pallas import tpu as pltpu
from jax.experimental.pallas import tpu_sc as plsc

HIDDEN = 64
OUT_DIM = 64
EPS = 1e-12

_Q = 256000
_AC = 5120
_NQB = _Q // _AC

_NC = 2
_NS = 16
_NW = _NC * _NS

_GSTEP = 640


def _table_g_body(x0_ref, x1_ref, x2_ref, x3_ref, wl_ref, blr_ref, o_ref):
    wl = wl_ref[...]
    blr = blr_ref[...]

    def gbits(x):
        u = jnp.mean(x, axis=0, keepdims=True)
        d = x - u
        s = jnp.mean(d * d, axis=0, keepdims=True)
        xn = d * lax.rsqrt(s + EPS)
        y = lax.dot_general(
            xn, wl, (((0,), (1,)), ((), ())),
            preferred_element_type=jnp.float32,
        ) + blr
        t = lax.bitcast_convert_type(y, jnp.int32)
        t = t + 0x7FFF + ((t >> 16) & 1)
        return (t >> 16) & 0xFFFF

    o_ref[:, 0:HIDDEN] = gbits(x0_ref[...]) | (gbits(x2_ref[...]) << 16)
    o_ref[:, HIDDEN:2 * HIDDEN] = gbits(x1_ref[...]) | (gbits(x3_ref[...]) << 16)


def _table_g(tableT, wl, bl):
    h, vocab = tableT.shape
    n3_last = (vocab - 3 * _Q - 1) // _AC
    return pl.pallas_call(
        _table_g_body,
        grid=(_NQB,),
        in_specs=[
            pl.BlockSpec((h, _AC), lambda i: (0, i)),
            pl.BlockSpec((h, _AC), lambda i: (0, _NQB + i)),
            pl.BlockSpec((h, _AC), lambda i: (0, 2 * _NQB + i)),
            pl.BlockSpec((h, _AC),
                         lambda i: (0, 3 * _NQB + jnp.minimum(i, n3_last))),
            pl.BlockSpec((h, h), lambda i: (0, 0)),
            pl.BlockSpec((1, h), lambda i: (0, 0)),
        ],
        out_specs=pl.BlockSpec((_AC, 2 * h), lambda i: (i, 0)),
        out_shape=jax.ShapeDtypeStruct((_Q, 2 * h), jnp.int32),
        compiler_params=pltpu.CompilerParams(fuse_transposed_lhs_in_matmul=True),
    )(tableT, tableT, tableT, tableT, wl, bl)


def _make_sc_gather(n_tok):
    per_w = n_tok // _NW
    step = _GSTEP
    n_outer = per_w // step
    assert per_w % step == 0
    mesh = plsc.VectorSubcoreMesh(core_axis_name="c", subcore_axis_name="s")

    @functools.partial(
        pl.kernel,
        mesh=mesh,
        out_type=jax.ShapeDtypeStruct((n_tok, HIDDEN), jnp.int32),
        scratch_types=[
            pltpu.VMEM((step,), jnp.int32),
            pltpu.VMEM((step,), jnp.int32),
            pltpu.VMEM((step, HIDDEN), jnp.int32),
            pltpu.VMEM((step, HIDDEN), jnp.int32),
            pltpu.SemaphoreType.DMA,
            pltpu.SemaphoreType.DMA,
        ],
        compiler_params=pltpu.CompilerParams(use_tc_tiling_on_sc=False),
    )
    def sc_gather(g_hbm, idx_hbm, out_hbm, idx_v0, idx_v1, rows_v0, rows_v1,
                  gsem, ssem):
        wid = lax.axis_index("s") * _NC + lax.axis_index("c")
        base = wid * per_w
        idx_bufs = (idx_v0, idx_v1)
        row_bufs = (rows_v0, rows_v1)
        pending = [None, None]
        for it in range(n_outer):
            p = it % 2
            off = base + it * step
            pltpu.sync_copy(idx_hbm.at[pl.ds(off, step)], idx_bufs[p])
            if pending[p] is not None:
                pending[p].wait()
            pltpu.async_copy(g_hbm.at[idx_bufs[p]], row_bufs[p], gsem).wait()
            pending[p] = pltpu.async_copy(
                row_bufs[p], out_hbm.at[pl.ds(off, step)], ssem)
        pending[0].wait()
        pending[1].wait()

    return sc_gather


def _sel_t_body(e_ref, q_ref, o_ref):
    e = e_ref[...]
    q = q_ref[...]
    lh = e.shape[0]

    def half(w, qh):
        qh3 = qh[:, :, None] >= 2
        bits = jnp.where(qh3, w & jnp.int32(-65536), w << 16)
        x = lax.bitcast_convert_type(bits, jnp.float32)
        return jnp.transpose(x, (0, 2, 1))

    lo = half(e[:, :, 0:HIDDEN], q[0:lh])
    hi = half(e[:, :, HIDDEN:2 * HIDDEN], q[lh:2 * lh])
    o_ref[...] = jnp.concatenate([lo, hi], axis=0)


def _sel_transpose(e4, q2, bb=256):
    lh, b, w = e4.shape
    l = 2 * lh
    assert b % bb == 0
    return pl.pallas_call(
        _sel_t_body,
        grid=(b // bb,),
        in_specs=[
            pl.BlockSpec((lh, bb, w), lambda i: (0, i, 0)),
            pl.BlockSpec((l, bb), lambda i: (0, i)),
        ],
        out_specs=pl.BlockSpec((l, OUT_DIM, bb), lambda i: (0, 0, i)),
        out_shape=jax.ShapeDtypeStruct((l, OUT_DIM, b), jnp.float32),
    )(e4, q2)


def kernel(flat_input_ids, tag_table, ln_weight, ln_bias, fc_w, fc_b):
    b, l = flat_input_ids.shape
    vocab, h = tag_table.shape
    n_tok = b * l

    ids = flat_input_ids.astype(jnp.int32)
    qt = (ids // _Q).T
    ridx = (2 * (ids % _Q) + ((ids // _Q) & 1)).T
    lh = l // 2
    idx_perm = jnp.stack([ridx[0:lh], ridx[lh:l]], axis=2).reshape(-1)

    wl = fc_w * ln_weight[None, :]
    bl = (fc_w @ ln_bias + fc_b).reshape(1, h)

    g = _table_g(tag_table.T, wl, bl)
    g4 = g.reshape(2 * _Q, h)
    e_g = _make_sc_gather(n_tok)(g4, idx_perm)
    e4 = e_g.reshape(lh, b, 128)
    out_t = _sel_transpose(e4, qt)
    return out_t.transpose(2, 0, 1)


# speedup vs baseline: 3.1254x; 1.1616x over previous
import functools

import jax
import jax.numpy as jnp
from jax import lax
from jax.experimental import pallas as pl
from jax.experimental.pallas import tpu as pltpu
from jax.experimental.pallas import tpu_sc as plsc

HIDDEN = 64
OUT_DIM = 64
EPS = 1e-12

_Q = 256000
_AC = 5120
_NQB = _Q // _AC

_NC = 2
_NS = 16
_NW = _NC * _NS

_GSTEP = 640


def _table_g_body(x0_ref, x1_ref, x2_ref, x3_ref, wl_ref, blr_ref, o_ref):
    wl = wl_ref[...]
    blr = blr_ref[...]

    def gbits(x):
        u = jnp.mean(x, axis=0, keepdims=True)
        d = x - u
        s = jnp.mean(d * d, axis=0, keepdims=True)
        xn = d * lax.rsqrt(s + EPS)
        y = lax.dot_general(
            xn, wl, (((0,), (1,)), ((), ())),
            preferred_element_type=jnp.float32,
        ) + blr
        t = lax.bitcast_convert_type(y, jnp.int32)
        t = t + 0x7FFF + ((t >> 16) & 1)
        return (t >> 16) & 0xFFFF

    o_ref[:, 0:HIDDEN] = gbits(x0_ref[...]) | (gbits(x2_ref[...]) << 16)
    o_ref[:, HIDDEN:2 * HIDDEN] = gbits(x1_ref[...]) | (gbits(x3_ref[...]) << 16)


def _table_g(tableT, wl, bl):
    h, vocab = tableT.shape
    n3_last = (vocab - 3 * _Q - 1) // _AC
    return pl.pallas_call(
        _table_g_body,
        grid=(_NQB,),
        in_specs=[
            pl.BlockSpec((h, _AC), lambda i: (0, i)),
            pl.BlockSpec((h, _AC), lambda i: (0, _NQB + i)),
            pl.BlockSpec((h, _AC), lambda i: (0, 2 * _NQB + i)),
            pl.BlockSpec((h, _AC),
                         lambda i: (0, 3 * _NQB + jnp.minimum(i, n3_last))),
            pl.BlockSpec((h, h), lambda i: (0, 0)),
            pl.BlockSpec((1, h), lambda i: (0, 0)),
        ],
        out_specs=pl.BlockSpec((_AC, 2 * h), lambda i: (i, 0)),
        out_shape=jax.ShapeDtypeStruct((_Q, 2 * h), jnp.int32),
        compiler_params=pltpu.CompilerParams(fuse_transposed_lhs_in_matmul=True),
    )(tableT, tableT, tableT, tableT, wl, bl)


def _make_sc_gather(n_tok):
    per_w = n_tok // _NW
    step = _GSTEP
    n_outer = per_w // step
    assert per_w % step == 0
    mesh = plsc.VectorSubcoreMesh(core_axis_name="c", subcore_axis_name="s")

    half = step // 2

    @functools.partial(
        pl.kernel,
        mesh=mesh,
        out_type=jax.ShapeDtypeStruct((n_tok, HIDDEN), jnp.int32),
        scratch_types=[
            pltpu.VMEM((half,), jnp.int32),
            pltpu.VMEM((half,), jnp.int32),
            pltpu.VMEM((step,), jnp.int32),
            pltpu.VMEM((step,), jnp.int32),
            pltpu.VMEM((step, HIDDEN), jnp.int32),
            pltpu.VMEM((step, HIDDEN), jnp.int32),
            pltpu.SemaphoreType.DMA,
            pltpu.SemaphoreType.DMA,
        ],
        compiler_params=pltpu.CompilerParams(
            use_tc_tiling_on_sc=False, needs_layout_passes=False),
    )
    def sc_gather(g_hbm, ilo_hbm, ihi_hbm, out_hbm, lo_v, hi_v,
                  int_v0, int_v1, rows_v0, rows_v1, gsem, ssem):
        wid = lax.axis_index("s") * _NC + lax.axis_index("c")
        base = wid * per_w
        int_bufs = (int_v0, int_v1)
        row_bufs = (rows_v0, rows_v1)
        lanes = lax.iota(jnp.int32, 16)
        pending = [None, None]
        for it in range(n_outer):
            p = it % 2
            off = base + it * step
            hoff = wid * (per_w // 2) + it * half
            pltpu.sync_copy(ilo_hbm.at[pl.ds(hoff, half)], lo_v)
            pltpu.sync_copy(ihi_hbm.at[pl.ds(hoff, half)], hi_v)
            for v in range(half // 16):
                dst = 2 * (v * 16 + lanes)
                plsc.store_scatter(int_bufs[p], [dst],
                                   lo_v[pl.ds(v * 16, 16)])
                plsc.store_scatter(int_bufs[p], [dst + 1],
                                   hi_v[pl.ds(v * 16, 16)])
            if pending[p] is not None:
                pending[p].wait()
            pltpu.async_copy(g_hbm.at[int_bufs[p]], row_bufs[p], gsem).wait()
            pending[p] = pltpu.async_copy(
                row_bufs[p], out_hbm.at[pl.ds(off, step)], ssem)
        pending[0].wait()
        pending[1].wait()

    return sc_gather


def _sel_t_body(e_ref, q_ref, o_ref):
    e = e_ref[...]
    q = q_ref[...]
    lh = e.shape[0]

    def half(w, qh):
        qh3 = qh[:, :, None] >= 2
        bits = jnp.where(qh3, w & jnp.int32(-65536), w << 16)
        x = lax.bitcast_convert_type(bits, jnp.float32)
        return jnp.transpose(x, (0, 2, 1))

    lo = half(e[:, :, 0:HIDDEN], q[0:lh])
    hi = half(e[:, :, HIDDEN:2 * HIDDEN], q[lh:2 * lh])
    o_ref[...] = jnp.concatenate([lo, hi], axis=0)


def _sel_transpose(e4, q2, bb=256):
    lh, b, w = e4.shape
    l = 2 * lh
    assert b % bb == 0
    return pl.pallas_call(
        _sel_t_body,
        grid=(b // bb,),
        in_specs=[
            pl.BlockSpec((lh, bb, w), lambda i: (0, i, 0)),
            pl.BlockSpec((l, bb), lambda i: (0, i)),
        ],
        out_specs=pl.BlockSpec((l, OUT_DIM, bb), lambda i: (0, 0, i)),
        out_shape=jax.ShapeDtypeStruct((l, OUT_DIM, b), jnp.float32),
    )(e4, q2)


def kernel(flat_input_ids, tag_table, ln_weight, ln_bias, fc_w, fc_b):
    b, l = flat_input_ids.shape
    vocab, h = tag_table.shape
    n_tok = b * l

    ids = flat_input_ids.astype(jnp.int32)
    qt = (ids // _Q).T
    ridx = (2 * (ids % _Q) + ((ids // _Q) & 1)).T
    lh = l // 2
    ilo = ridx[0:lh].reshape(-1)
    ihi = ridx[lh:l].reshape(-1)

    wl = fc_w * ln_weight[None, :]
    bl = (fc_w @ ln_bias + fc_b).reshape(1, h)

    g = _table_g(tag_table.T, wl, bl)
    g4 = g.reshape(2 * _Q, h)
    e_g = _make_sc_gather(n_tok)(g4, ilo, ihi)
    e4 = e_g.reshape(lh, b, 128)
    out_t = _sel_transpose(e4, qt)
    return out_t.transpose(2, 0, 1)
